# baseline (speedup 1.0000x reference)
import functools
import jax
import jax.numpy as jnp
from jax import lax
from jax.experimental import pallas as pl
from jax.experimental.pallas import tpu as pltpu
from jax.experimental.pallas import tpu_sc as plsc


def _argmin_body(x_ref, w_in_t_ref, b_in_ref, embed_t_ref, embed_ref,
                 w_out_t_ref, b_out_ref, idx_ref, eout_ref):
    z = jnp.dot(x_ref[...], w_in_t_ref[...],
                preferred_element_type=jnp.float32) + b_in_ref[...]
    et = embed_t_ref[...]
    s = jnp.dot(z, et, preferred_element_type=jnp.float32)
    e_sq = jnp.sum(et * et, axis=0, keepdims=True)
    scores = 2.0 * s - e_sq
    m = jnp.max(scores, axis=1, keepdims=True)
    k = scores.shape[1]
    iota = jax.lax.broadcasted_iota(jnp.int32, scores.shape, 1)
    idx = jnp.min(jnp.where(scores == m, iota, k), axis=1)
    idx_ref[...] = idx.reshape(idx_ref.shape)

    @pl.when(pl.program_id(0) == 0)
    def _():
        eout_ref[...] = jnp.dot(embed_ref[...], w_out_t_ref[...],
                                preferred_element_type=jnp.float32) + b_out_ref[...]


def _make_sc_gather(bt, dim, n_workers, chunk):
    b_per_w = bt // n_workers
    n_chunks = b_per_w // chunk
    mesh = plsc.VectorSubcoreMesh(core_axis_name="c", subcore_axis_name="s")
    nc = mesh.num_cores

    @functools.partial(
        pl.kernel,
        out_type=jax.ShapeDtypeStruct((bt, dim), jnp.float32),
        mesh=mesh,
        scratch_types=[
            pltpu.VMEM((chunk,), jnp.int32),
            pltpu.VMEM((chunk, dim), jnp.float32),
            pltpu.SemaphoreType.DMA,
        ],
    )
    def sc_gather(idx_hbm, table_hbm, out_hbm, idx_v, rows_v, sem):
        wid = lax.axis_index("s") * nc + lax.axis_index("c")
        base = wid * b_per_w
        for c in range(n_chunks):
            off = base + c * chunk
            pltpu.sync_copy(idx_hbm.at[pl.ds(off, chunk)], idx_v)
            pltpu.async_copy(table_hbm.at[idx_v], rows_v, sem).wait()
            pltpu.sync_copy(rows_v, out_hbm.at[pl.ds(off, chunk)])

    return sc_gather


@jax.jit
def kernel(x, W_in, b_in, W_out, b_out, embed):
    b, t, dim = x.shape
    cdim, _ = W_in.shape
    k = embed.shape[0]
    bt = b * t
    flat = x.reshape(bt, dim)
    r = 512
    nt = bt // r

    idx3, e_out = pl.pallas_call(
        _argmin_body,
        grid=(nt,),
        in_specs=[
            pl.BlockSpec((r, dim), lambda i: (i, 0)),
            pl.BlockSpec((dim, cdim), lambda i: (0, 0)),
            pl.BlockSpec((1, cdim), lambda i: (0, 0)),
            pl.BlockSpec((cdim, k), lambda i: (0, 0)),
            pl.BlockSpec((k, cdim), lambda i: (0, 0)),
            pl.BlockSpec((cdim, dim), lambda i: (0, 0)),
            pl.BlockSpec((1, dim), lambda i: (0, 0)),
        ],
        out_specs=[
            pl.BlockSpec((1, 1, r), lambda i: (i, 0, 0)),
            pl.BlockSpec((k, dim), lambda i: (0, 0)),
        ],
        out_shape=[
            jax.ShapeDtypeStruct((nt, 1, r), jnp.int32),
            jax.ShapeDtypeStruct((k, dim), jnp.float32),
        ],
    )(flat, W_in.T, b_in.reshape(1, cdim), embed.T, embed,
      W_out.T, b_out.reshape(1, dim))

    idx = idx3.reshape(bt)
    info = plsc.get_sparse_core_info()
    n_workers = info.num_cores * info.num_subcores
    out = _make_sc_gather(bt, dim, n_workers, 128)(idx, e_out)
    return out.reshape(b, t, dim)
